# baseline (speedup 1.0000x reference)
import functools

import jax
import jax.numpy as jnp
from jax import lax
from jax.experimental import pallas as pl
from jax.experimental.pallas import tpu as pltpu
from jax.experimental.pallas import tpu_sc as plsc

BATCH = 16384
DIM = 16
NC = 2
NS = 16
NW = NC * NS
B_PER_W = BATCH // NW
CHUNK = 128
N_CHUNK = B_PER_W // CHUNK
GRP = 16
N_GRP = B_PER_W // GRP


def _lane_perm(x, idx):
    return lax.gather(
        x, idx[:, None],
        lax.GatherDimensionNumbers(
            offset_dims=(), collapsed_slice_dims=(0,), start_index_map=(0,)),
        (1,), mode=lax.GatherScatterMode.PROMISE_IN_BOUNDS)


def _body(u_hbm, v_hbm, table_hbm, out_hbm, idx_u, idx_v, rows_u, rows_v,
          out_buf, sem):
    wid = lax.axis_index("s") * NC + lax.axis_index("c")

    pltpu.sync_copy(u_hbm.at[wid], idx_u)
    pltpu.sync_copy(v_hbm.at[wid], idx_v)

    copies = []
    for c in range(N_CHUNK):
        sl = pl.ds(c * CHUNK, CHUNK)
        copies.append(
            pltpu.async_copy(table_hbm.at[idx_u.at[sl]], rows_u.at[sl], sem))
        copies.append(
            pltpu.async_copy(table_hbm.at[idx_v.at[sl]], rows_v.at[sl], sem))
    for cp in copies:
        cp.wait()

    lanes = lax.iota(jnp.int32, GRP)

    def grp_step(g, carry):
        base = g * GRP
        r = jnp.zeros((GRP,), jnp.float32)
        for j in range(GRP):
            ur = rows_u[base + j, :]
            vr = rows_v[base + j, :]
            x = ur * vr
            for sh in (8, 4, 2, 1):
                x = x + _lane_perm(x, lanes ^ sh)
            r = jnp.where(lanes == j, x, r)
        out_buf[pl.ds(base, GRP)] = 1.0 / (1.0 + jnp.exp(-r))
        return carry

    lax.fori_loop(0, N_GRP, grp_step, 0)

    pltpu.sync_copy(out_buf, out_hbm.at[wid])


@jax.jit
def _run(u2, v2, embed_weight):
    mesh = plsc.VectorSubcoreMesh(core_axis_name="c", subcore_axis_name="s")
    kfn = pl.kernel(
        _body,
        out_type=jax.ShapeDtypeStruct((NW, B_PER_W), jnp.float32),
        mesh=mesh,
        scratch_types=[
            pltpu.VMEM((B_PER_W,), jnp.int32),
            pltpu.VMEM((B_PER_W,), jnp.int32),
            pltpu.VMEM((B_PER_W, DIM), jnp.float32),
            pltpu.VMEM((B_PER_W, DIM), jnp.float32),
            pltpu.VMEM((B_PER_W,), jnp.float32),
            pltpu.SemaphoreType.DMA,
        ],
        compiler_params=pltpu.CompilerParams(use_tc_tiling_on_sc=False),
    )
    return kfn(u2, v2, embed_weight)


def kernel(u, v, embed_weight):
    u2 = u.astype(jnp.int32).reshape(NW, B_PER_W)
    v2 = v.astype(jnp.int32).reshape(NW, B_PER_W)
    out = _run(u2, v2, embed_weight)
    return out.reshape(BATCH)
